# baseline (speedup 1.0000x reference)
import functools

import jax
import jax.numpy as jnp
from jax import lax
from jax.experimental import pallas as pl
from jax.experimental.pallas import tpu as pltpu
from jax.experimental.pallas import tpu_sc as plsc

NBUF = 4


@functools.lru_cache(maxsize=None)
def _build(batch, hist, vocab, dim):
    mesh = plsc.VectorSubcoreMesh(core_axis_name="c", subcore_axis_name="s")
    nw = mesh.num_cores * mesh.num_subcores
    assert hist % 8 == 0 and batch % 128 == 0 and dim % 8 == 0
    ha, btn, dtn = hist // 8, batch // 128, dim // 8
    units = hist * btn
    assert units % nw == 0
    upw = units // nw
    assert upw % NBUF == 0 and upw >= 2 * NBUF

    @functools.partial(
        pl.kernel,
        mesh=mesh,
        out_type=jax.ShapeDtypeStruct((hist, dtn, btn, 8, 128), jnp.float32),
        scratch_types=[
            pltpu.VMEM((NBUF, 128), jnp.int32),
            pltpu.VMEM((NBUF, 128, dim), jnp.float32),
            pltpu.VMEM((NBUF, dtn, 8, 128), jnp.float32),
            [pltpu.SemaphoreType.DMA] * NBUF,
            [pltpu.SemaphoreType.DMA] * NBUF,
            [pltpu.SemaphoreType.DMA] * NBUF,
        ],
        compiler_params=pltpu.CompilerParams(
            use_tc_tiling_on_sc=False, needs_layout_passes=False
        ),
    )
    def k(idx4_hbm, table_hbm, out5_hbm, idx_v, rows_v, trans_v,
          isem, gsem, osem):
        wid = lax.axis_index("s") * mesh.num_cores + lax.axis_index("c")
        u0 = wid * upw

        def unit_coords(u):
            ug = u0 + u
            h = lax.shift_right_logical(ug, 7)
            bt = lax.bitwise_and(ug, 127)
            return lax.shift_right_logical(h, 3), lax.bitwise_and(h, 7), h, bt

        def fire_idx(u, b):
            a, e, _, bt = unit_coords(u)
            pltpu.async_copy(idx4_hbm.at[a, bt, e], idx_v.at[b], isem[b])

        def drain_idx(b):
            pltpu.make_async_copy(
                idx4_hbm.at[0, 0, 0], idx_v.at[b], isem[b]
            ).wait()

        def fire_gather(u, b):
            pltpu.async_copy(
                table_hbm.at[idx_v.at[b]], rows_v.at[b], gsem[b]
            )

        def drain_gather(b):
            pltpu.make_async_copy(
                table_hbm.at[pl.ds(0, 128)], rows_v.at[b], gsem[b]
            ).wait()

        def transpose(b):
            base = lax.iota(jnp.int32, 16)
            rows = rows_v.at[b]
            for dt in range(dtn):
                for ds in range(8):
                    cid = jnp.full((16,), dt * 8 + ds, jnp.int32)
                    for l in range(8):
                        v = plsc.load_gather(rows, [base + l * 16, cid])
                        trans_v[b, dt, ds, pl.ds(l * 16, 16)] = v

        def fire_out(u, b):
            _, _, h, bt = unit_coords(u)
            for dt in range(dtn):
                pltpu.async_copy(
                    trans_v.at[b, dt], out5_hbm.at[h, dt, bt], osem[b]
                )

        def drain_out(b):
            for dt in range(dtn):
                pltpu.make_async_copy(
                    trans_v.at[b, dt], out5_hbm.at[0, dt, 0], osem[b]
                ).wait()

        def body(u, b, first, guard):
            b2 = (b + 2) % NBUF
            drain_gather(b)
            transpose(b)
            fire_out(u, b)
            if guard:
                @pl.when(u + 4 < upw)
                def _():
                    fire_idx(u + 4, b)
            else:
                fire_idx(u + 4, b)
            if not first:
                drain_out(b2)
            drain_idx(b2)
            fire_gather(u + 2, b2)

        for b in range(NBUF):
            fire_idx(b, b)
        drain_idx(0)
        fire_gather(0, 0)
        drain_idx(1)
        fire_gather(1, 1)
        body(0, 0, first=True, guard=False)
        body(1, 1, first=True, guard=False)

        def step(s, carry):
            t = NBUF * s + 2
            for o in range(NBUF):
                body(t + o, (2 + o) % NBUF, first=False, guard=True)
            return carry

        lax.fori_loop(0, (upw - 4) // NBUF, step, 0)

        for u in (upw - 2, upw - 1):
            b = u % NBUF
            drain_gather(b)
            transpose(b)
            fire_out(u, b)
        for b in range(NBUF):
            drain_out(b)

    return k


def kernel(inputs, table):
    batch, hist = inputs.shape
    vocab, dim = table.shape
    x4 = (
        inputs.astype(jnp.int32)
        .T.reshape(hist // 8, 8, batch // 128, 128)
        .transpose(0, 2, 1, 3)
    )
    out5 = _build(batch, hist, vocab, dim)(x4, table)
    return out5.transpose(2, 4, 0, 1, 3).reshape(batch, hist, dim)


# speedup vs baseline: 1.4660x; 1.4660x over previous
import functools

import jax
import jax.numpy as jnp
from jax import lax
from jax.experimental import pallas as pl
from jax.experimental.pallas import tpu as pltpu
from jax.experimental.pallas import tpu_sc as plsc

NBUF = 4


@functools.lru_cache(maxsize=None)
def _build(batch, hist, vocab, dim):
    mesh = plsc.VectorSubcoreMesh(core_axis_name="c", subcore_axis_name="s")
    nw = mesh.num_cores * mesh.num_subcores
    assert hist % 8 == 0 and batch % 128 == 0 and dim % 8 == 0
    ha, btn, dtn = hist // 8, batch // 128, dim // 8
    units = hist * btn
    assert units % nw == 0
    upw = units // nw
    assert upw % NBUF == 0 and upw >= 2 * NBUF

    @functools.partial(
        pl.kernel,
        mesh=mesh,
        out_type=jax.ShapeDtypeStruct((hist, dtn, btn, 1024), jnp.float32),
        scratch_types=[
            pltpu.VMEM((NBUF, 128), jnp.int32),
            pltpu.VMEM((NBUF, 128, dim), jnp.float32),
            pltpu.VMEM((NBUF, dim * 128), jnp.float32),
            [pltpu.SemaphoreType.DMA] * NBUF,
            [pltpu.SemaphoreType.DMA] * NBUF,
            [pltpu.SemaphoreType.DMA] * NBUF,
        ],
        compiler_params=pltpu.CompilerParams(
            use_tc_tiling_on_sc=False, needs_layout_passes=False
        ),
    )
    def k(idx4_hbm, table_hbm, out5_hbm, idx_v, rows_v, trans_v,
          isem, gsem, osem):
        wid = lax.axis_index("s") * mesh.num_cores + lax.axis_index("c")
        u0 = wid * upw

        def unit_coords(u):
            ug = u0 + u
            h = lax.shift_right_logical(ug, 7)
            bt = lax.bitwise_and(ug, 127)
            return lax.shift_right_logical(h, 3), lax.bitwise_and(h, 7), h, bt

        def fire_idx(u, b):
            a, e, _, bt = unit_coords(u)
            pltpu.async_copy(idx4_hbm.at[a, bt, e], idx_v.at[b], isem[b])

        def drain_idx(b):
            pltpu.make_async_copy(
                idx4_hbm.at[0, 0, 0], idx_v.at[b], isem[b]
            ).wait()

        def fire_gather(u, b):
            pltpu.async_copy(
                table_hbm.at[idx_v.at[b]], rows_v.at[b], gsem[b]
            )

        def drain_gather(b):
            pltpu.make_async_copy(
                table_hbm.at[pl.ds(0, 128)], rows_v.at[b], gsem[b]
            ).wait()

        def transpose(b):
            base = lax.iota(jnp.int32, 16)
            rows = rows_v.at[b]

            @plsc.parallel_loop(0, dim * 8, 1, unroll=8)
            def _(i):
                d = lax.shift_right_logical(i, 3)
                l16 = lax.bitwise_and(i, 7) * 16
                v = plsc.load_gather(
                    rows, [base + l16, jnp.full((16,), 0, jnp.int32) + d]
                )
                trans_v[b, pl.ds(d * 128 + l16, 16)] = v

        def fire_out(u, b):
            _, _, h, bt = unit_coords(u)
            for dt in range(dtn):
                pltpu.async_copy(
                    trans_v.at[b, pl.ds(dt * 1024, 1024)],
                    out5_hbm.at[h, dt, bt],
                    osem[b],
                )

        def drain_out(b):
            for dt in range(dtn):
                pltpu.make_async_copy(
                    trans_v.at[b, pl.ds(dt * 1024, 1024)],
                    out5_hbm.at[0, dt, 0],
                    osem[b],
                ).wait()

        def body(u, b, first, guard):
            b2 = (b + 2) % NBUF
            drain_gather(b)
            transpose(b)
            fire_out(u, b)
            if guard:
                @pl.when(u + 4 < upw)
                def _():
                    fire_idx(u + 4, b)
            else:
                fire_idx(u + 4, b)
            if not first:
                drain_out(b2)
            drain_idx(b2)
            fire_gather(u + 2, b2)

        for b in range(NBUF):
            fire_idx(b, b)
        drain_idx(0)
        fire_gather(0, 0)
        drain_idx(1)
        fire_gather(1, 1)
        body(0, 0, first=True, guard=False)
        body(1, 1, first=True, guard=False)

        def step(s, carry):
            t = NBUF * s + 2
            for o in range(NBUF):
                body(t + o, (2 + o) % NBUF, first=False, guard=True)
            return carry

        lax.fori_loop(0, (upw - 4) // NBUF, step, 0)

        for u in (upw - 2, upw - 1):
            b = u % NBUF
            drain_gather(b)
            transpose(b)
            fire_out(u, b)
        for b in range(NBUF):
            drain_out(b)

    return k


def kernel(inputs, table):
    batch, hist = inputs.shape
    vocab, dim = table.shape
    x4 = (
        inputs.astype(jnp.int32)
        .T.reshape(hist // 8, 8, batch // 128, 128)
        .transpose(0, 2, 1, 3)
    )
    out5 = _build(batch, hist, vocab, dim)(x4, table)
    out5 = out5.reshape(hist, dim // 8, batch // 128, 8, 128)
    return out5.transpose(2, 4, 0, 1, 3).reshape(batch, hist, dim)


# speedup vs baseline: 3.3449x; 2.2817x over previous
import functools

import jax
import jax.numpy as jnp
from jax import lax
from jax.experimental import pallas as pl
from jax.experimental.pallas import tpu as pltpu
from jax.experimental.pallas import tpu_sc as plsc

NBUF = 4


@functools.lru_cache(maxsize=None)
def _build(batch, hist, vocab, dim):
    mesh = plsc.VectorSubcoreMesh(core_axis_name="c", subcore_axis_name="s")
    nw = mesh.num_cores * mesh.num_subcores
    assert hist % 8 == 0 and batch % 128 == 0 and dim % 8 == 0
    ha, btn, dtn = hist // 8, batch // 128, dim // 8
    units = hist * btn
    assert units % nw == 0
    upw = units // nw
    assert upw % NBUF == 0 and upw >= 2 * NBUF

    @functools.partial(
        pl.kernel,
        mesh=mesh,
        out_type=jax.ShapeDtypeStruct((hist, dtn, btn, 1024), jnp.float32),
        scratch_types=[
            pltpu.VMEM((NBUF, 128), jnp.int32),
            pltpu.VMEM((NBUF, 128, dim), jnp.float32),
            pltpu.VMEM((NBUF, dim * 128), jnp.float32),
            [pltpu.SemaphoreType.DMA] * NBUF,
            [pltpu.SemaphoreType.DMA] * NBUF,
            [pltpu.SemaphoreType.DMA] * NBUF,
        ],
        compiler_params=pltpu.CompilerParams(
            use_tc_tiling_on_sc=False, needs_layout_passes=False
        ),
    )
    def k(idx4_hbm, table_hbm, out5_hbm, idx_v, rows_v, trans_v,
          isem, gsem, osem):
        wid = lax.axis_index("s") * mesh.num_cores + lax.axis_index("c")
        u0 = wid * upw

        def unit_coords(u):
            ug = u0 + u
            h = lax.shift_right_logical(ug, 7)
            bt = lax.bitwise_and(ug, 127)
            return lax.shift_right_logical(h, 3), lax.bitwise_and(h, 7), h, bt

        def fire_idx(u, b):
            a, e, _, bt = unit_coords(u)
            pltpu.async_copy(idx4_hbm.at[a, bt, e], idx_v.at[b], isem[b])

        def drain_idx(b):
            pltpu.make_async_copy(
                idx4_hbm.at[0, 0, 0], idx_v.at[b], isem[b]
            ).wait()

        def fire_gather(u, b):
            pltpu.async_copy(
                table_hbm.at[idx_v.at[b]], rows_v.at[b], gsem[b]
            )

        def drain_gather(b):
            pltpu.make_async_copy(
                table_hbm.at[pl.ds(0, 128)], rows_v.at[b], gsem[b]
            ).wait()

        def transpose(b):
            base = lax.iota(jnp.int32, 16)
            rows = rows_v.at[b]

            @plsc.parallel_loop(0, dim * 8, 1, unroll=8)
            def _(i):
                d0 = lax.shift_right_logical(i, 3)
                l16 = lax.bitwise_and(i, 7) * 16
                dvec = lax.bitwise_and(d0 + base, dim - 1)
                v = plsc.load_gather(rows, [base + l16, dvec])
                plsc.store_scatter(
                    trans_v.at[b], [dvec * 128 + l16 + base], v
                )

        def fire_out(u, b):
            _, _, h, bt = unit_coords(u)
            for dt in range(dtn):
                pltpu.async_copy(
                    trans_v.at[b, pl.ds(dt * 1024, 1024)],
                    out5_hbm.at[h, dt, bt],
                    osem[b],
                )

        def drain_out(b):
            for dt in range(dtn):
                pltpu.make_async_copy(
                    trans_v.at[b, pl.ds(dt * 1024, 1024)],
                    out5_hbm.at[0, dt, 0],
                    osem[b],
                ).wait()

        def body(u, b, first, guard):
            b2 = (b + 2) % NBUF
            drain_gather(b)
            transpose(b)
            fire_out(u, b)
            if guard:
                @pl.when(u + 4 < upw)
                def _():
                    fire_idx(u + 4, b)
            else:
                fire_idx(u + 4, b)
            if not first:
                drain_out(b2)
            drain_idx(b2)
            fire_gather(u + 2, b2)

        for b in range(NBUF):
            fire_idx(b, b)
        drain_idx(0)
        fire_gather(0, 0)
        drain_idx(1)
        fire_gather(1, 1)
        body(0, 0, first=True, guard=False)
        body(1, 1, first=True, guard=False)

        def step(s, carry):
            t = NBUF * s + 2
            for o in range(NBUF):
                body(t + o, (2 + o) % NBUF, first=False, guard=True)
            return carry

        lax.fori_loop(0, (upw - 4) // NBUF, step, 0)

        for u in (upw - 2, upw - 1):
            b = u % NBUF
            drain_gather(b)
            transpose(b)
            fire_out(u, b)
        for b in range(NBUF):
            drain_out(b)

    return k


def kernel(inputs, table):
    batch, hist = inputs.shape
    vocab, dim = table.shape
    x4 = (
        inputs.astype(jnp.int32)
        .T.reshape(hist // 8, 8, batch // 128, 128)
        .transpose(0, 2, 1, 3)
    )
    out5 = _build(batch, hist, vocab, dim)(x4, table)
    out5 = out5.reshape(hist, dim // 8, batch // 128, 8, 128)
    return out5.transpose(2, 4, 0, 1, 3).reshape(batch, hist, dim)


# speedup vs baseline: 3.4557x; 1.0331x over previous
import functools

import jax
import jax.numpy as jnp
from jax import lax
from jax.experimental import pallas as pl
from jax.experimental.pallas import tpu as pltpu
from jax.experimental.pallas import tpu_sc as plsc

NBUF = 4


@functools.lru_cache(maxsize=None)
def _build(batch, hist, vocab, dim):
    mesh = plsc.VectorSubcoreMesh(core_axis_name="c", subcore_axis_name="s")
    nw = mesh.num_cores * mesh.num_subcores
    assert hist % 8 == 0 and batch % 128 == 0 and dim % 8 == 0
    ha, btn, dtn = hist // 8, batch // 128, dim // 8
    units = hist * btn
    assert units % nw == 0
    upw = units // nw
    assert upw % NBUF == 0 and upw >= 2 * NBUF

    @functools.partial(
        pl.kernel,
        mesh=mesh,
        out_type=jax.ShapeDtypeStruct((hist, dtn, btn, 1024), jnp.float32),
        scratch_types=[
            pltpu.VMEM((NBUF, 128), jnp.int32),
            pltpu.VMEM((NBUF, 128, dim), jnp.float32),
            pltpu.VMEM((NBUF, dim * 128), jnp.float32),
            [pltpu.SemaphoreType.DMA] * NBUF,
            [pltpu.SemaphoreType.DMA] * NBUF,
            [pltpu.SemaphoreType.DMA] * NBUF,
        ],
        compiler_params=pltpu.CompilerParams(
            use_tc_tiling_on_sc=False, needs_layout_passes=False
        ),
    )
    def k(idx4_hbm, table_hbm, out5_hbm, idx_v, rows_v, trans_v,
          isem, gsem, osem):
        wid = lax.axis_index("s") * mesh.num_cores + lax.axis_index("c")
        u0 = wid * upw

        def unit_coords(u):
            ug = u0 + u
            h = lax.shift_right_logical(ug, 7)
            bt = lax.bitwise_and(ug, 127)
            return lax.shift_right_logical(h, 3), lax.bitwise_and(h, 7), h, bt

        def fire_idx(u, b):
            a, e, _, bt = unit_coords(u)
            pltpu.async_copy(idx4_hbm.at[a, bt, e], idx_v.at[b], isem[b])

        def drain_idx(b):
            pltpu.make_async_copy(
                idx4_hbm.at[0, 0, 0], idx_v.at[b], isem[b]
            ).wait()

        def fire_gather(u, b):
            pltpu.async_copy(
                table_hbm.at[idx_v.at[b]], rows_v.at[b], gsem[b]
            )

        def drain_gather(b):
            pltpu.make_async_copy(
                table_hbm.at[pl.ds(0, 128)], rows_v.at[b], gsem[b]
            ).wait()

        def transpose(b):
            base = lax.iota(jnp.int32, 16)
            rows = rows_v.at[b]

            @plsc.parallel_loop(0, dim * 8, 1, unroll=16)
            def _(i):
                d0 = lax.shift_right_logical(i, 3)
                l16 = lax.bitwise_and(i, 7) * 16
                dvec = lax.bitwise_and(d0 + base, dim - 1)
                v = plsc.load_gather(rows, [base + l16, dvec])
                plsc.store_scatter(
                    trans_v.at[b], [dvec * 128 + l16 + base], v
                )

        def fire_out(u, b):
            _, _, h, bt = unit_coords(u)
            for dt in range(dtn):
                pltpu.async_copy(
                    trans_v.at[b, pl.ds(dt * 1024, 1024)],
                    out5_hbm.at[h, dt, bt],
                    osem[b],
                )

        def drain_out(b):
            for dt in range(dtn):
                pltpu.make_async_copy(
                    trans_v.at[b, pl.ds(dt * 1024, 1024)],
                    out5_hbm.at[0, dt, 0],
                    osem[b],
                ).wait()

        def body(u, b, first, guard):
            b2 = (b + 2) % NBUF
            drain_gather(b)
            transpose(b)
            fire_out(u, b)
            if guard:
                @pl.when(u + 4 < upw)
                def _():
                    fire_idx(u + 4, b)
            else:
                fire_idx(u + 4, b)
            if not first:
                drain_out(b2)
            drain_idx(b2)
            fire_gather(u + 2, b2)

        for b in range(NBUF):
            fire_idx(b, b)
        drain_idx(0)
        fire_gather(0, 0)
        drain_idx(1)
        fire_gather(1, 1)
        body(0, 0, first=True, guard=False)
        body(1, 1, first=True, guard=False)

        def step(s, carry):
            t = NBUF * s + 2
            for o in range(NBUF):
                body(t + o, (2 + o) % NBUF, first=False, guard=True)
            return carry

        lax.fori_loop(0, (upw - 4) // NBUF, step, 0)

        for u in (upw - 2, upw - 1):
            b = u % NBUF
            drain_gather(b)
            transpose(b)
            fire_out(u, b)
        for b in range(NBUF):
            drain_out(b)

    return k


def kernel(inputs, table):
    batch, hist = inputs.shape
    vocab, dim = table.shape
    x4 = (
        inputs.astype(jnp.int32)
        .T.reshape(hist // 8, 8, batch // 128, 128)
        .transpose(0, 2, 1, 3)
    )
    out5 = _build(batch, hist, vocab, dim)(x4, table)
    out5 = out5.reshape(hist, dim // 8, batch // 128, 8, 128)
    return out5.transpose(2, 4, 0, 1, 3).reshape(batch, hist, dim)
